# baseline (speedup 1.0000x reference)
import jax
import jax.numpy as jnp
from jax import lax
from jax.experimental import pallas as pl
from jax.experimental.pallas import tpu as pltpu
from jax.experimental.pallas import tpu_sc as plsc

VOCAB = 128
DIM = 32
SEQ = 200
ROWS_PER_STEP = 4


def kernel(x, table):
    nrows, seq = x.shape
    idx = x.reshape(1, nrows * seq).astype(jnp.int32)
    tab_flat = table.astype(jnp.float32).reshape(VOCAB * DIM)

    mesh = plsc.VectorSubcoreMesh(core_axis_name="core",
                                  subcore_axis_name="subcore")
    ngroups = (seq + 15) // 16

    @pl.kernel(out_type=jax.ShapeDtypeStruct((nrows, seq, DIM), jnp.float32),
               mesh=mesh,
               compiler_params=pltpu.CompilerParams(
                   use_tc_tiling_on_sc=False, needs_layout_passes=False),
               scratch_types=[pltpu.VMEM((VOCAB * DIM,), jnp.float32)])
    def gather_kernel(table_hbm, i_hbm, o_hbm, tab_v):
        pltpu.sync_copy(table_hbm, tab_v)
        lanes = lax.iota(jnp.int32, 16)

        def body(i_vmem, o_vmem):
            @pl.loop(0, ROWS_PER_STEP)
            def _(r):
                @pl.loop(0, ngroups)
                def _(g):
                    j0 = jnp.minimum(g * 16, seq - 16)
                    vbase = i_vmem[0, pl.ds(r * seq + j0, 16)] * DIM
                    for u in range(16):
                        a0 = vbase[u] + lanes
                        o_vmem[r, j0 + u, pl.ds(0, 16)] = plsc.load_gather(
                            tab_v, [a0])
                        o_vmem[r, j0 + u, pl.ds(16, 16)] = plsc.load_gather(
                            tab_v, [a0 + 16])

        pltpu.emit_pipeline(
            body,
            grid=(nrows // ROWS_PER_STEP,),
            in_specs=[pl.BlockSpec((1, ROWS_PER_STEP * seq),
                                   lambda i: (0, i))],
            out_specs=[pl.BlockSpec((ROWS_PER_STEP, seq, DIM),
                                    lambda i: (i, 0, 0))],
            core_axis_name=("core", "subcore"),
            dimension_semantics=(pltpu.PARALLEL,),
        )(i_hbm, o_hbm)

    return gather_kernel(tab_flat, idx)
